# baseline (speedup 1.0000x reference)
import functools

import jax
import jax.numpy as jnp
from jax import lax
from jax.experimental import pallas as pl
from jax.experimental.pallas import tpu as pltpu
from jax.experimental.pallas import tpu_sc as plsc

B = 16384
D = 16
L = 16
NC = 2
NS = 16
NW = NC * NS
BPW = B // NW
CH = 128
NCH = BPW // CH
GPC = CH // L
SR = 128


def _sc_partials(x, su, bu, sv, bv, ht, vt):
    mesh = plsc.VectorSubcoreMesh(core_axis_name="c", subcore_axis_name="s")

    @functools.partial(
        pl.kernel,
        mesh=mesh,
        compiler_params=pltpu.CompilerParams(needs_layout_passes=False),
        out_type=jax.ShapeDtypeStruct((NW, D), jnp.float32),
        scratch_types=[
            pltpu.VMEM((BPW,), jnp.int32),
            pltpu.VMEM((BPW,), jnp.int32),
            pltpu.VMEM((BPW,), jnp.int32),
            pltpu.VMEM((BPW,), jnp.int32),
            pltpu.VMEM((CH, SR), jnp.float32),
            pltpu.VMEM((CH, SR), jnp.float32),
            pltpu.VMEM((CH, SR), jnp.float32),
            pltpu.VMEM((CH, SR), jnp.float32),
            pltpu.VMEM((BPW // 8, SR), jnp.float32),
            pltpu.VMEM((D,), jnp.float32),
            pltpu.SemaphoreType.DMA,
            pltpu.SemaphoreType.DMA,
            pltpu.SemaphoreType.DMA,
            pltpu.SemaphoreType.DMA,
            pltpu.SemaphoreType.DMA,
        ],
    )
    def k(x_hbm, su_hbm, bu_hbm, sv_hbm, bv_hbm, ht_hbm, vt_hbm, out_hbm,
          su_v, bu_v, sv_v, bv_v, u0, u1, v0, v1, x_v, acc_v,
          semu0, semu1, semv0, semv1, semx):
        wid = lax.axis_index("s") * NC + lax.axis_index("c")
        base = wid * BPW
        iota = lax.iota(jnp.int32, L)
        ubufs = (u0, u1)
        vbufs = (v0, v1)
        usems = (semu0, semu1)
        vsems = (semv0, semv1)

        pltpu.sync_copy(su_hbm.at[pl.ds(base, BPW)], su_v)
        pltpu.sync_copy(sv_hbm.at[pl.ds(base, BPW)], sv_v)

        def fire(c):
            sl = pl.ds(c * CH, CH)
            pltpu.async_copy(ht_hbm.at[su_v.at[sl]], ubufs[c % 2], usems[c % 2])
            pltpu.async_copy(vt_hbm.at[sv_v.at[sl]], vbufs[c % 2], vsems[c % 2])

        fire(0)
        cx = pltpu.async_copy(x_hbm.at[pl.ds(wid * (BPW // 8), BPW // 8)],
                              x_v, semx)
        pltpu.sync_copy(bu_hbm.at[pl.ds(base, BPW)], bu_v)
        pltpu.sync_copy(bv_hbm.at[pl.ds(base, BPW)], bv_v)
        cx.wait()

        zero = jnp.zeros((L,), jnp.float32)
        ts = (zero,) * D

        for c in range(NCH):
            if c + 1 < NCH:
                fire(c + 1)
            pltpu.make_async_copy(ht_hbm.at[pl.ds(0, CH)], ubufs[c % 2],
                                  usems[c % 2]).wait()
            pltpu.make_async_copy(vt_hbm.at[pl.ds(0, CH)], vbufs[c % 2],
                                  vsems[c % 2]).wait()
            ub = ubufs[c % 2]
            vb = vbufs[c % 2]

            def group(g, ts, c=c, ub=ub, vb=vb):
                rloc = g * L + iota
                xsup = (c * (CH // 8) + g * (L // 8)) + (iota >> 3)
                xcb = (iota & 7) * D
                subu = bu_v[pl.dslice(c * CH + g * L, L)] * D
                subv = bv_v[pl.dslice(c * CH + g * L, L)] * D
                s = zero
                for kk in range(D):
                    uc = plsc.load_gather(ub, [rloc, subu + kk])
                    xc = plsc.load_gather(x_v, [xsup, xcb + kk])
                    s = s + uc * xc
                new_ts = []
                for kk in range(D):
                    vc = plsc.load_gather(vb, [rloc, subv + kk])
                    new_ts.append(ts[kk] + s * vc)
                return tuple(new_ts)

            ts = lax.fori_loop(jnp.int32(0), jnp.int32(GPC), group, ts)

        acc = jnp.zeros((L,), jnp.float32)
        for kk in range(D):
            onehot = (iota == kk).astype(jnp.float32)
            acc = acc + jnp.sum(ts[kk]) * onehot
        acc_v[...] = acc
        pltpu.sync_copy(acc_v, out_hbm.at[wid])

    return k(x, su, bu, sv, bv, ht, vt)


def _tc_body(y_ref, p_ref, o_ref):
    t = jnp.sum(p_ref[...], axis=0, keepdims=True)
    o_ref[...] = jnp.sum(y_ref[...] * t, axis=1)


def _tc_finish(y, partials):
    return pl.pallas_call(
        _tc_body,
        out_shape=jax.ShapeDtypeStruct((B,), jnp.float32),
    )(y, partials)


def kernel(x, y, x_nodes, y_nodes, human_table, virus_table):
    xn = x_nodes.astype(jnp.int32)
    yn = y_nodes.astype(jnp.int32)
    su = xn >> 3
    bu = xn & 7
    sv = yn >> 3
    bv = yn & 7
    ht = human_table.reshape(-1, SR)
    vt = virus_table.reshape(-1, SR)
    x128 = x.reshape(-1, SR)
    partials = _sc_partials(x128, su, bu, sv, bv, ht, vt)
    out = _tc_finish(y, partials)
    return out.astype(jnp.float64)


# speedup vs baseline: 1.4370x; 1.4370x over previous
import functools

import jax
import jax.numpy as jnp
from jax import lax
from jax.experimental import pallas as pl
from jax.experimental.pallas import tpu as pltpu
from jax.experimental.pallas import tpu_sc as plsc

B = 16384
D = 16
L = 16
NC = 2
NS = 16
NW = NC * NS
BPW = B // NW
CH = 128
NCH = BPW // CH
GPC = CH // L
SR = 128


def _sc_partials(x, xn, sv, bv, ht, vt):
    mesh = plsc.VectorSubcoreMesh(core_axis_name="c", subcore_axis_name="s")

    @functools.partial(
        pl.kernel,
        mesh=mesh,
        compiler_params=pltpu.CompilerParams(needs_layout_passes=False),
        out_type=jax.ShapeDtypeStruct((NW, D), jnp.float32),
        scratch_types=[
            pltpu.VMEM((BPW,), jnp.int32),
            pltpu.VMEM((BPW,), jnp.int32),
            pltpu.VMEM((BPW,), jnp.int32),
            pltpu.VMEM((CH, D), jnp.float32),
            pltpu.VMEM((CH, D), jnp.float32),
            pltpu.VMEM((CH, SR), jnp.float32),
            pltpu.VMEM((CH, SR), jnp.float32),
            pltpu.VMEM((BPW // 8, SR), jnp.float32),
            pltpu.VMEM((D,), jnp.float32),
            pltpu.SemaphoreType.DMA,
            pltpu.SemaphoreType.DMA,
            pltpu.SemaphoreType.DMA,
            pltpu.SemaphoreType.DMA,
            pltpu.SemaphoreType.DMA,
        ],
    )
    def k(x_hbm, xn_hbm, sv_hbm, bv_hbm, ht_hbm, vt_hbm, out_hbm,
          iu_v, sv_v, bv_v, u0, u1, v0, v1, x_v, acc_v,
          semu0, semu1, semv0, semv1, semx):
        wid = lax.axis_index("s") * NC + lax.axis_index("c")
        base = wid * BPW
        iota = lax.iota(jnp.int32, L)
        ubufs = (u0, u1)
        vbufs = (v0, v1)
        usems = (semu0, semu1)
        vsems = (semv0, semv1)

        pltpu.sync_copy(xn_hbm.at[pl.ds(base, BPW)], iu_v)
        pltpu.sync_copy(sv_hbm.at[pl.ds(base, BPW)], sv_v)

        def fire(c):
            sl = pl.ds(c * CH, CH)
            pltpu.async_copy(vt_hbm.at[sv_v.at[sl]], vbufs[c % 2], vsems[c % 2])
            ub = ubufs[c % 2]
            su = usems[c % 2]

            def issue(g, carry):
                vx = iu_v[pl.ds(c * CH + g * L, L)]
                for j in range(L):
                    r = g * L + j
                    pltpu.async_copy(ht_hbm.at[pl.ds(vx[j], 1)],
                                     ub.at[pl.ds(r, 1)], su)
                return carry
            lax.fori_loop(jnp.int32(0), jnp.int32(GPC), issue, 0)

        fire(0)
        cx = pltpu.async_copy(x_hbm.at[pl.ds(wid * (BPW // 8), BPW // 8)],
                              x_v, semx)
        pltpu.sync_copy(bv_hbm.at[pl.ds(base, BPW)], bv_v)
        cx.wait()

        zero = jnp.zeros((L,), jnp.float32)
        ts = (zero,) * D

        for c in range(NCH):
            if c + 1 < NCH:
                fire(c + 1)
            pltpu.make_async_copy(ht_hbm.at[pl.ds(0, CH)], ubufs[c % 2],
                                  usems[c % 2]).wait()
            pltpu.make_async_copy(vt_hbm.at[pl.ds(0, CH)], vbufs[c % 2],
                                  vsems[c % 2]).wait()
            ub = ubufs[c % 2]
            vb = vbufs[c % 2]

            def group(g, ts, c=c, ub=ub, vb=vb):
                rloc = g * L + iota
                xsup = (c * (CH // 8) + g * (L // 8)) + (iota >> 3)
                xcb = (iota & 7) * D
                subv = bv_v[pl.dslice(c * CH + g * L, L)] * D
                s = zero
                for kk in range(D):
                    kvec = jnp.full((L,), kk, jnp.int32)
                    uc = plsc.load_gather(ub, [rloc, kvec])
                    xc = plsc.load_gather(x_v, [xsup, xcb + kk])
                    s = s + uc * xc
                new_ts = []
                for kk in range(D):
                    vc = plsc.load_gather(vb, [rloc, subv + kk])
                    new_ts.append(ts[kk] + s * vc)
                return tuple(new_ts)

            ts = lax.fori_loop(jnp.int32(0), jnp.int32(GPC), group, ts)

        acc = jnp.zeros((L,), jnp.float32)
        for kk in range(D):
            onehot = (iota == kk).astype(jnp.float32)
            acc = acc + jnp.sum(ts[kk]) * onehot
        acc_v[...] = acc
        pltpu.sync_copy(acc_v, out_hbm.at[wid])

    return k(x, xn, sv, bv, ht, vt)


def _tc_body(y_ref, p_ref, o_ref):
    t = jnp.sum(p_ref[...], axis=0, keepdims=True)
    o_ref[...] = jnp.sum(y_ref[...] * t, axis=1)


def _tc_finish(y, partials):
    return pl.pallas_call(
        _tc_body,
        out_shape=jax.ShapeDtypeStruct((B,), jnp.float32),
    )(y, partials)


def kernel(x, y, x_nodes, y_nodes, human_table, virus_table):
    xn = x_nodes.astype(jnp.int32)
    yn = y_nodes.astype(jnp.int32)
    sv = yn >> 3
    bv = yn & 7
    vt = virus_table.reshape(-1, SR)
    x128 = x.reshape(-1, SR)
    partials = _sc_partials(x128, xn, sv, bv, human_table, vt)
    out = _tc_finish(y, partials)
    return out.astype(jnp.float64)
